# Initial kernel scaffold; baseline (speedup 1.0000x reference)
import jax
import jax.numpy as jnp
from jax.experimental import pallas as pl


def kernel(x, coeffs, importance):
    raise NotImplementedError("write your pallas kernel here")


# speedup vs baseline: 14.7659x; 14.7659x over previous
import functools

import jax
import jax.numpy as jnp
from jax.experimental import pallas as pl
from jax.experimental.pallas import tpu as pltpu

_D = 2048
_KEEP = max(1, int(_D * 0.5))
_ROWS_PER_BLOCK = 512


def _mask_kernel(imp_row_ref, imp_col_ref, out_ref):
    imp_col = imp_col_ref[:, :]
    e_idx = jax.lax.broadcasted_iota(jnp.int32, (_D, 1), 0)
    chunk = 256
    for c in range(_D // chunk):
        d_vals = imp_row_ref[0:1, c * chunk:(c + 1) * chunk]
        d_idx = jax.lax.broadcasted_iota(jnp.int32, (1, chunk), 1) + c * chunk
        gt = jnp.sum((imp_col > d_vals).astype(jnp.float32), axis=0,
                     keepdims=True)
        eq_before = jnp.sum(
            ((imp_col == d_vals) & (e_idx < d_idx)).astype(jnp.float32),
            axis=0, keepdims=True)
        out_ref[0:1, c * chunk:(c + 1) * chunk] = (
            (gt + eq_before) < float(_KEEP)).astype(jnp.float32)


def _poly_kernel(coef_ref, mask_ref, x_ref, o_ref):
    x = x_ref[...]
    c0 = coef_ref[0]
    c1 = coef_ref[1]
    c2 = coef_ref[2]
    p = x * (c0 + x * (c1 + x * c2))
    m = mask_ref[0:1, :]
    o_ref[...] = jnp.where(m != 0.0, p, x)


@jax.jit
def kernel(x, coeffs, importance):
    B, T, D = x.shape
    assert D == _D

    mask = pl.pallas_call(
        _mask_kernel,
        out_shape=jax.ShapeDtypeStruct((1, D), jnp.float32),
    )(importance.reshape(1, D), importance.reshape(D, 1))

    xf = x.reshape(B * T, D)
    n_blocks = (B * T) // _ROWS_PER_BLOCK
    out = pl.pallas_call(
        _poly_kernel,
        grid=(n_blocks,),
        in_specs=[
            pl.BlockSpec(memory_space=pltpu.SMEM),
            pl.BlockSpec((1, D), lambda i: (0, 0)),
            pl.BlockSpec((_ROWS_PER_BLOCK, D), lambda i: (i, 0)),
        ],
        out_specs=pl.BlockSpec((_ROWS_PER_BLOCK, D), lambda i: (i, 0)),
        out_shape=jax.ShapeDtypeStruct((B * T, D), jnp.float32),
    )(coeffs, mask, xf)
    return out.reshape(B, T, D)


# speedup vs baseline: 15.0959x; 1.0224x over previous
import functools

import jax
import jax.numpy as jnp
from jax.experimental import pallas as pl
from jax.experimental.pallas import tpu as pltpu

_D = 2048
_KEEP = max(1, int(_D * 0.5))
_ROWS_PER_BLOCK = 1024


def _mask_kernel(imp_row_ref, imp_col_ref, out_ref):
    imp_col = imp_col_ref[:, :]
    e_idx = jax.lax.broadcasted_iota(jnp.int32, (_D, 1), 0)
    chunk = 256
    for c in range(_D // chunk):
        d_vals = imp_row_ref[0:1, c * chunk:(c + 1) * chunk]
        d_idx = jax.lax.broadcasted_iota(jnp.int32, (1, chunk), 1) + c * chunk
        gt = jnp.sum((imp_col > d_vals).astype(jnp.float32), axis=0,
                     keepdims=True)
        eq_before = jnp.sum(
            ((imp_col == d_vals) & (e_idx < d_idx)).astype(jnp.float32),
            axis=0, keepdims=True)
        out_ref[0:1, c * chunk:(c + 1) * chunk] = (
            (gt + eq_before) < float(_KEEP)).astype(jnp.float32)


def _poly_kernel(coef_ref, mask_ref, x_ref, o_ref):
    x = x_ref[...]
    c0 = coef_ref[0]
    c1 = coef_ref[1]
    c2 = coef_ref[2]
    p = x * (c0 + x * (c1 + x * c2))
    m = mask_ref[0:1, :]
    o_ref[...] = jnp.where(m != 0.0, p, x)


@jax.jit
def kernel(x, coeffs, importance):
    B, T, D = x.shape
    assert D == _D

    mask = pl.pallas_call(
        _mask_kernel,
        out_shape=jax.ShapeDtypeStruct((1, D), jnp.float32),
    )(importance.reshape(1, D), importance.reshape(D, 1))

    xf = x.reshape(B * T, D)
    n_blocks = (B * T) // _ROWS_PER_BLOCK
    out = pl.pallas_call(
        _poly_kernel,
        grid=(n_blocks,),
        in_specs=[
            pl.BlockSpec(memory_space=pltpu.SMEM),
            pl.BlockSpec((1, D), lambda i: (0, 0)),
            pl.BlockSpec((_ROWS_PER_BLOCK, D), lambda i: (i, 0)),
        ],
        out_specs=pl.BlockSpec((_ROWS_PER_BLOCK, D), lambda i: (i, 0)),
        out_shape=jax.ShapeDtypeStruct((B * T, D), jnp.float32),
    )(coeffs, mask, xf)
    return out.reshape(B, T, D)


# speedup vs baseline: 15.3361x; 1.0159x over previous
import jax
import jax.numpy as jnp
from jax.experimental import pallas as pl
from jax.experimental.pallas import tpu as pltpu

_D = 2048
_KEEP = max(1, int(_D * 0.5))
_ROWS_PER_BLOCK = 1024


def _poly_kernel(coef_ref, imp_row_ref, imp_col_ref, x_ref, o_ref, mask_ref):
    @pl.when(pl.program_id(0) == 0)
    def _compute_mask():
        imp_col = imp_col_ref[:, :]
        e_idx = jax.lax.broadcasted_iota(jnp.int32, (_D, 1), 0)
        chunk = 256
        for c in range(_D // chunk):
            d_vals = imp_row_ref[0:1, c * chunk:(c + 1) * chunk]
            d_idx = jax.lax.broadcasted_iota(jnp.int32, (1, chunk), 1) + c * chunk
            gt = jnp.sum((imp_col > d_vals).astype(jnp.float32), axis=0,
                         keepdims=True)
            eq_before = jnp.sum(
                ((imp_col == d_vals) & (e_idx < d_idx)).astype(jnp.float32),
                axis=0, keepdims=True)
            mask_ref[0:1, c * chunk:(c + 1) * chunk] = (
                (gt + eq_before) < float(_KEEP)).astype(jnp.float32)

    x = x_ref[...]
    c0 = coef_ref[0]
    c1 = coef_ref[1]
    c2 = coef_ref[2]
    p = x * (c0 + x * (c1 + x * c2))
    m = mask_ref[0:1, :]
    o_ref[...] = jnp.where(m != 0.0, p, x)


@jax.jit
def kernel(x, coeffs, importance):
    B, T, D = x.shape
    assert D == _D

    xf = x.reshape(B * T, D)
    n_blocks = (B * T) // _ROWS_PER_BLOCK
    out = pl.pallas_call(
        _poly_kernel,
        grid=(n_blocks,),
        in_specs=[
            pl.BlockSpec(memory_space=pltpu.SMEM),
            pl.BlockSpec((1, D), lambda i: (0, 0)),
            pl.BlockSpec((D, 1), lambda i: (0, 0)),
            pl.BlockSpec((_ROWS_PER_BLOCK, D), lambda i: (i, 0)),
        ],
        out_specs=pl.BlockSpec((_ROWS_PER_BLOCK, D), lambda i: (i, 0)),
        out_shape=jax.ShapeDtypeStruct((B * T, D), jnp.float32),
        scratch_shapes=[pltpu.VMEM((1, D), jnp.float32)],
    )(coeffs, importance.reshape(1, D), importance.reshape(D, 1), xf)
    return out.reshape(B, T, D)
